# baseline (speedup 1.0000x reference)
import functools
import math

import jax
import jax.numpy as jnp
from jax import lax
from jax.experimental import pallas as pl
from jax.experimental.pallas import tpu as pltpu
from jax.experimental.pallas import tpu_sc as plsc

DIM = 1024
SCALE = math.sqrt(DIM)
LANES = 16
NC, NS = 2, 16
NW = NC * NS
CHUNK = 32
NBUF = 3
GROUP = 32


def _body(n_chunks, idx_hbm, table_hbm, out_hbm, idx_v, bufs, gsems, ssems):
    wid = lax.axis_index("s") * NC + lax.axis_index("c")
    rows_per_worker = n_chunks * CHUNK
    base = wid * rows_per_worker

    w_per_row = idx_hbm.shape[1] // rows_per_worker
    pltpu.sync_copy(
        idx_hbm.at[wid // w_per_row, pl.ds((wid % w_per_row) * rows_per_worker, rows_per_worker)],
        idx_v,
    )

    def g_desc(c, b):
        return pltpu.make_async_copy(
            table_hbm.at[idx_v.at[pl.ds(c * CHUNK, CHUNK)]], bufs[b], gsems[b]
        )

    def s_desc(c, b):
        return pltpu.make_async_copy(
            bufs[b], out_hbm.at[pl.ds(base + c * CHUNK, CHUNK)], ssems[b]
        )

    def scale(b):
        def row_body(r, carry):
            def col_body(j, carry2):
                sl = pl.ds(j * LANES, LANES)
                bufs[b][r, sl] = bufs[b][r, sl] * SCALE
                return carry2

            return lax.fori_loop(0, DIM // LANES, col_body, carry, unroll=8)

        lax.fori_loop(0, CHUNK, row_body, 0)

    g_desc(0, 0).start()
    g_desc(1, 1).start()

    g_desc(0, 0).wait()
    scale(0)
    g_desc(2, 2).start()
    s_desc(0, 0).start()

    g_desc(1, 1).wait()
    scale(1)
    s_desc(0, 0).wait()
    g_desc(3, 0).start()
    s_desc(1, 1).start()

    def round_body(r, carry):
        for p in range(3):
            c = 2 + 3 * r + p
            bp = (2 + p) % 3
            pbuf = (bp + 2) % 3
            g_desc(c, bp).wait()
            scale(bp)
            s_desc(c - 1, pbuf).wait()

            @pl.when(c + 2 < n_chunks)
            def _():
                g_desc(c + 2, pbuf).start()

            s_desc(c, bp).start()
        return carry

    lax.fori_loop(0, (n_chunks - 2) // 3, round_body, 0)
    s_desc(n_chunks - 1, (n_chunks - 1) % NBUF).wait()


def kernel(x, table):
    batch = x.size
    idx = x if x.dtype == jnp.int32 else x.astype(jnp.int32)
    rows_per_worker = batch // NW
    n_chunks = rows_per_worker // CHUNK

    mesh = plsc.VectorSubcoreMesh(core_axis_name="c", subcore_axis_name="s")
    run = functools.partial(
        pl.kernel,
        out_type=jax.ShapeDtypeStruct((batch, DIM), jnp.float32),
        mesh=mesh,
        compiler_params=pltpu.CompilerParams(use_tc_tiling_on_sc=False),
        scratch_types=[
            pltpu.VMEM((rows_per_worker,), jnp.int32),
            tuple(pltpu.VMEM((CHUNK, DIM), jnp.float32) for _ in range(NBUF)),
            tuple(pltpu.SemaphoreType.DMA for _ in range(NBUF)),
            tuple(pltpu.SemaphoreType.DMA for _ in range(NBUF)),
        ],
    )(functools.partial(_body, n_chunks))

    out = run(idx, table)
    return out.reshape(x.shape + (DIM,))


# speedup vs baseline: 7.7172x; 7.7172x over previous
import functools
import math

import jax
import jax.numpy as jnp
from jax import lax
from jax.experimental import pallas as pl
from jax.experimental.pallas import tpu as pltpu
from jax.experimental.pallas import tpu_sc as plsc

DIM = 1024
SCALE = math.sqrt(DIM)
LANES = 16
NC, NS = 2, 16
NW = NC * NS
CHUNK = 32
NBUF = 3
GROUP = 32


def _body(n_chunks, idx_hbm, table_hbm, out_hbm, idx_v, bufs, gsems, ssems):
    wid = lax.axis_index("s") * NC + lax.axis_index("c")
    rows_per_worker = n_chunks * CHUNK
    base = wid * rows_per_worker

    w_per_row = idx_hbm.shape[1] // rows_per_worker
    pltpu.sync_copy(
        idx_hbm.at[wid // w_per_row, pl.ds((wid % w_per_row) * rows_per_worker, rows_per_worker)],
        idx_v,
    )

    def g_desc(c, b):
        return pltpu.make_async_copy(
            table_hbm.at[idx_v.at[pl.ds(c * CHUNK, CHUNK)]], bufs[b], gsems[b]
        )

    def s_desc(c, b):
        return pltpu.make_async_copy(
            bufs[b], out_hbm.at[pl.ds(base + c * CHUNK, CHUNK)], ssems[b]
        )

    def scale(b):
        def row_body(r, carry):
            def col_body(j, carry2):
                sl = pl.ds(j * LANES, LANES)
                bufs[b][r, sl] = bufs[b][r, sl] * SCALE
                return carry2

            return lax.fori_loop(0, DIM // LANES, col_body, carry, unroll=8)

        lax.fori_loop(0, CHUNK, row_body, 0)

    g_desc(0, 0).start()
    g_desc(1, 1).start()

    g_desc(0, 0).wait()
    scale(0)
    g_desc(2, 2).start()
    s_desc(0, 0).start()

    g_desc(1, 1).wait()
    scale(1)
    s_desc(0, 0).wait()
    g_desc(3, 0).start()
    s_desc(1, 1).start()

    def round_body(r, carry):
        for p in range(3):
            c = 2 + 3 * r + p
            bp = (2 + p) % 3
            pbuf = (bp + 2) % 3
            g_desc(c, bp).wait()
            scale(bp)
            s_desc(c - 1, pbuf).wait()

            @pl.when(c + 2 < n_chunks)
            def _():
                g_desc(c + 2, pbuf).start()

            s_desc(c, bp).start()
        return carry

    lax.fori_loop(0, (n_chunks - 2) // 3, round_body, 0)
    s_desc(n_chunks - 1, (n_chunks - 1) % NBUF).wait()


def kernel(x, table):
    batch = x.size
    idx = x if x.dtype == jnp.int32 else x.astype(jnp.int32)
    rows_per_worker = batch // NW
    n_chunks = rows_per_worker // CHUNK

    mesh = plsc.VectorSubcoreMesh(core_axis_name="c", subcore_axis_name="s")
    run = functools.partial(
        pl.kernel,
        out_type=jax.ShapeDtypeStruct((batch, DIM), jnp.float32),
        mesh=mesh,
        compiler_params=pltpu.CompilerParams(skip_device_barrier=True),
        scratch_types=[
            pltpu.VMEM((rows_per_worker,), jnp.int32),
            tuple(pltpu.VMEM((CHUNK, DIM), jnp.float32) for _ in range(NBUF)),
            tuple(pltpu.SemaphoreType.DMA for _ in range(NBUF)),
            tuple(pltpu.SemaphoreType.DMA for _ in range(NBUF)),
        ],
    )(functools.partial(_body, n_chunks))

    out = run(idx, table)
    return out.reshape(x.shape + (DIM,))


# speedup vs baseline: 7.7443x; 1.0035x over previous
import functools
import math

import jax
import jax.numpy as jnp
from jax import lax
from jax.experimental import pallas as pl
from jax.experimental.pallas import tpu as pltpu
from jax.experimental.pallas import tpu_sc as plsc

DIM = 1024
SCALE = math.sqrt(DIM)
LANES = 16
NC, NS = 2, 16
NW = NC * NS
CHUNK = 32
NBUF = 3
GROUP = 32


def _body(n_chunks, idx_hbm, table_hbm, out_hbm, idx_v, bufs, gsems, ssems):
    wid = lax.axis_index("s") * NC + lax.axis_index("c")
    rows_per_worker = n_chunks * CHUNK
    base = wid * rows_per_worker

    w_per_row = idx_hbm.shape[1] // rows_per_worker
    pltpu.sync_copy(
        idx_hbm.at[wid // w_per_row, pl.ds((wid % w_per_row) * rows_per_worker, rows_per_worker)],
        idx_v,
    )

    def g_desc(c, b):
        return pltpu.make_async_copy(
            table_hbm.at[idx_v.at[pl.ds(c * CHUNK, CHUNK)]], bufs[b], gsems[b]
        )

    def s_desc(c, b):
        return pltpu.make_async_copy(
            bufs[b], out_hbm.at[pl.ds(base + c * CHUNK, CHUNK)], ssems[b]
        )

    def scale(b):
        def row_body(r, carry):
            def col_body(j, carry2):
                sl = pl.ds(j * LANES, LANES)
                bufs[b][r, sl] = bufs[b][r, sl] * SCALE
                return carry2

            return lax.fori_loop(0, DIM // LANES, col_body, carry, unroll=8)

        lax.fori_loop(0, CHUNK, row_body, 0)

    g_desc(0, 0).start()
    g_desc(1, 1).start()

    g_desc(0, 0).wait()
    scale(0)
    g_desc(2, 2).start()
    s_desc(0, 0).start()

    g_desc(1, 1).wait()
    scale(1)
    s_desc(0, 0).wait()
    g_desc(3, 0).start()
    s_desc(1, 1).start()

    def round_body(r, carry):
        for p in range(3):
            c = 2 + 3 * r + p
            bp = (2 + p) % 3
            pbuf = (bp + 2) % 3
            g_desc(c, bp).wait()
            scale(bp)
            s_desc(c - 1, pbuf).wait()

            @pl.when(c + 2 < n_chunks)
            def _():
                g_desc(c + 2, pbuf).start()

            s_desc(c, bp).start()
        return carry

    lax.fori_loop(0, (n_chunks - 2) // 3, round_body, 0)
    s_desc(n_chunks - 1, (n_chunks - 1) % NBUF).wait()


def kernel(x, table):
    batch = x.size
    idx = x if x.dtype == jnp.int32 else x.astype(jnp.int32)
    rows_per_worker = batch // NW
    n_chunks = rows_per_worker // CHUNK

    mesh = plsc.VectorSubcoreMesh(core_axis_name="c", subcore_axis_name="s")
    run = functools.partial(
        pl.kernel,
        out_type=jax.ShapeDtypeStruct((batch, DIM), jnp.float32),
        mesh=mesh,
        scratch_types=[
            pltpu.VMEM((rows_per_worker,), jnp.int32),
            tuple(pltpu.VMEM((CHUNK, DIM), jnp.float32) for _ in range(NBUF)),
            tuple(pltpu.SemaphoreType.DMA for _ in range(NBUF)),
            tuple(pltpu.SemaphoreType.DMA for _ in range(NBUF)),
        ],
    )(functools.partial(_body, n_chunks))

    out = run(idx, table)
    return out.reshape(x.shape + (DIM,))
